# baseline (speedup 1.0000x reference)
import functools
import jax
import jax.numpy as jnp
from jax import lax
from jax.experimental import pallas as pl
from jax.experimental.pallas import tpu as pltpu
from jax.experimental.pallas import tpu_sc as plsc

NFEAT = 128
HIDDEN = 128
NLAYERS = 3
K = 3
EFEAT = 2

_NW = 32
_C = 128


def _spmv_calls(NP, EP):
    n_chunks = EP // (_NW * _C)
    rps = NP // 16
    mesh = plsc.VectorSubcoreMesh(core_axis_name="c", subcore_axis_name="s")

    @functools.partial(
        pl.kernel, mesh=mesh,
        out_type=jax.ShapeDtypeStruct((2 * NP, NFEAT), jnp.float32),
        scratch_types=[
            pltpu.VMEM((4, _C), jnp.int32),
            pltpu.VMEM((4, _C), jnp.int32),
            pltpu.VMEM((4, _C), jnp.float32),
            pltpu.VMEM((2, _C, NFEAT), jnp.float32),
            pltpu.VMEM_SHARED((NP, NFEAT), jnp.float32),
            pltpu.SemaphoreType.DMA,
            pltpu.SemaphoreType.DMA,
            pltpu.SemaphoreType.DMA,
        ],
    )
    def spmv(y_h, g_h, s_h, w_h, out_h, gi, si, wv, rows, acc,
             sem_i, sem_g, sem_s):
        cid = lax.axis_index("c")
        sid = lax.axis_index("s")
        wid = sid * 2 + cid
        base = wid * n_chunks

        def zrow(r, c):
            for f in range(NFEAT // 16):
                rows[0, r, pl.ds(f * 16, 16)] = jnp.zeros((16,), jnp.float32)
            return c

        lax.fori_loop(0, _C, zrow, 0)
        for kblk in range(rps // _C):
            pltpu.sync_copy(rows.at[0],
                            acc.at[pl.ds(sid * rps + kblk * _C, _C)])
        plsc.subcore_barrier()

        def idx_issue(ci, r4):
            o = (base + ci) * _C
            pltpu.async_copy(g_h.at[pl.ds(o, _C)], gi.at[r4], sem_i)
            pltpu.async_copy(s_h.at[pl.ds(o, _C)], si.at[r4], sem_i)
            pltpu.async_copy(w_h.at[pl.ds(o, _C)], wv.at[r4], sem_i)

        def idx_wait(ci, r4):
            o = (base + ci) * _C
            pltpu.make_async_copy(g_h.at[pl.ds(o, _C)], gi.at[r4],
                                  sem_i).wait()
            pltpu.make_async_copy(s_h.at[pl.ds(o, _C)], si.at[r4],
                                  sem_i).wait()
            pltpu.make_async_copy(w_h.at[pl.ds(o, _C)], wv.at[r4],
                                  sem_i).wait()

        idx_issue(0, 0)
        idx_wait(0, 0)
        pltpu.async_copy(y_h.at[gi.at[0]], rows.at[0], sem_g)
        idx_issue(1, 1)

        def chunk(ci, carry):
            b = lax.rem(ci, 2)
            r4 = lax.rem(ci, 4)
            r4n = lax.rem(ci + 1, 4)
            r4p = lax.rem(ci + 3, 4)

            @pl.when(ci + 1 < n_chunks)
            def _():
                idx_wait(ci + 1, r4n)

                @pl.when(ci >= 1)
                def _():
                    pltpu.make_async_copy(
                        rows.at[1 - b], acc.at[si.at[r4p]], sem_s).wait()

                pltpu.async_copy(y_h.at[gi.at[r4n]], rows.at[1 - b], sem_g)

            pltpu.make_async_copy(y_h.at[gi.at[r4]], rows.at[b],
                                  sem_g).wait()

            def group(g, c2):
                wrow = wv[r4, pl.ds(g * 16, 16)]
                for i in range(16):
                    ws = lax.gather(
                        wrow, jnp.full((16, 1), i, jnp.int32),
                        lax.GatherDimensionNumbers(
                            offset_dims=(), collapsed_slice_dims=(0,),
                            start_index_map=(0,)),
                        slice_sizes=(1,),
                        mode=lax.GatherScatterMode.PROMISE_IN_BOUNDS)
                    e = g * 16 + i
                    for f in range(NFEAT // 16):
                        seg = rows[b, e, pl.ds(f * 16, 16)]
                        rows[b, e, pl.ds(f * 16, 16)] = seg * ws
                return c2

            lax.fori_loop(0, _C // 16, group, 0)
            pltpu.async_copy(rows.at[b], acc.at[si.at[r4]], sem_s, add=True)

            @pl.when(ci + 2 < n_chunks)
            def _():
                idx_issue(ci + 2, lax.rem(ci + 2, 4))

            return carry

        lax.fori_loop(0, n_chunks, chunk, 0)
        lastb = lax.rem(n_chunks - 1, 2)
        last4 = lax.rem(n_chunks - 1, 4)
        pltpu.make_async_copy(rows.at[lastb], acc.at[si.at[last4]],
                              sem_s).wait()
        plsc.subcore_barrier()
        pltpu.sync_copy(acc.at[pl.ds(sid * rps, rps)],
                        out_h.at[pl.ds(cid * NP + sid * rps, rps)])

    return spmv


def _flag_body(ei_ref, e0_ref, o_ref):
    i = pl.program_id(0)

    @pl.when(i == 0)
    def _():
        o_ref[...] = jnp.zeros_like(o_ref)

    r0 = e0_ref[0, 0]
    c0 = e0_ref[1, 0]
    hit = jnp.logical_and(ei_ref[0:1, :] == c0, ei_ref[1:2, :] == r0)
    hitf = jnp.max(hit.astype(jnp.float32), axis=1, keepdims=True)
    o_ref[...] = jnp.maximum(o_ref[...], hitf)


def _mlp_body(efT_ref, ei_ref, W1t_ref, b1_ref, W2t_ref, b2_ref, found_ref,
              wf_ref, wr_ref):
    ef = efT_ref[...]
    h = jnp.dot(W1t_ref[...], ef, preferred_element_type=jnp.float32)
    h = jnp.maximum(h + b1_ref[...], 0.0)
    t = jnp.dot(W2t_ref[...], h, preferred_element_type=jnp.float32)
    w = ef + t + b2_ref[...]
    loop = ei_ref[0:1, :] == ei_ref[1:2, :]
    w = jnp.where(loop, 0.0, w)
    wf_ref[...] = w
    wr_ref[...] = w * (1.0 - found_ref[0, 0])


def _prep_body(d0a, d0b, d1a, d1b, o0, o1):
    o0[...] = d0a[...] + d0b[...] - 1.0
    o1[...] = d1a[...] + d1b[...] - 1.0


def _stage_a_body(h, dm0, dm1, p0a, p0b, p1a, p1b, t0, t1):
    hv = h[...]
    t0[...] = dm0[...] * hv - p0a[...] - p0b[...]
    t1[...] = dm1[...] * hv - p1a[...] - p1b[...]


def _stage_b_body(h, t0, t1, dm0, dm1, q0a, q0b, q1a, q1b, W0, W1, bias, out,
                  *, relu):
    hv = h[...]
    t0v = t0[...]
    t1v = t1[...]
    x2_0 = 2.0 * (dm0[...] * t0v - q0a[...] - q0b[...]) - hv
    x2_1 = 2.0 * (dm1[...] * t1v - q1a[...] - q1b[...]) - hv
    dot = lambda a, b: jnp.dot(a, b, preferred_element_type=jnp.float32)
    acc = dot(hv, W0[0] + W1[0])
    acc = acc + dot(t0v, W0[1]) + dot(t1v, W1[1])
    acc = acc + dot(x2_0, W0[2]) + dot(x2_1, W1[2])
    acc = acc + bias[0:1, :] + bias[1:2, :]
    if relu:
        acc = jnp.maximum(acc, 0.0)
    out[...] = acc


def kernel(x, edge_attr, et_W1, et_b1, et_W2, et_b2, cheb_W, cheb_b, edge_index):
    N = x.shape[0]
    E = edge_index.shape[1]
    NP = ((N + 255) // 256) * 256
    EP = ((2 * E + _NW * _C * 8 - 1) // (_NW * _C * 8)) * (_NW * _C * 8)
    f32 = jnp.float32

    ei = edge_index.astype(jnp.int32)
    row, col = ei[0], ei[1]

    BE = 2560
    ge = E // BE
    e0 = ei[:, 0:1]
    found = pl.pallas_call(
        _flag_body,
        grid=(ge,),
        in_specs=[
            pl.BlockSpec((2, BE), lambda i: (0, i)),
            pl.BlockSpec(memory_space=pltpu.SMEM),
        ],
        out_specs=pl.BlockSpec((1, 1), lambda i: (0, 0)),
        out_shape=jax.ShapeDtypeStruct((1, 1), f32),
    )(ei, e0)

    efT = edge_attr[:, :EFEAT].T
    W1t = et_W1.T
    b1c = et_b1[:, None]
    W2t = et_W2.T
    b2c = et_b2[:, None]
    wf, wr = pl.pallas_call(
        _mlp_body,
        grid=(ge,),
        in_specs=[
            pl.BlockSpec((2, BE), lambda i: (0, i)),
            pl.BlockSpec((2, BE), lambda i: (0, i)),
            pl.BlockSpec((HIDDEN, 2), lambda i: (0, 0)),
            pl.BlockSpec((HIDDEN, 1), lambda i: (0, 0)),
            pl.BlockSpec((2, HIDDEN), lambda i: (0, 0)),
            pl.BlockSpec((2, 1), lambda i: (0, 0)),
            pl.BlockSpec(memory_space=pltpu.SMEM),
        ],
        out_specs=[
            pl.BlockSpec((2, BE), lambda i: (0, i)),
            pl.BlockSpec((2, BE), lambda i: (0, i)),
        ],
        out_shape=[
            jax.ShapeDtypeStruct((2, E), f32),
            jax.ShapeDtypeStruct((2, E), f32),
        ],
    )(efT, ei, W1t, b1c, W2t, b2c, found)

    pad = EP - 2 * E
    zi = jnp.zeros((pad,), jnp.int32)
    zf = jnp.zeros((pad,), f32)
    rows2 = jnp.concatenate([row, col, zi])
    cols2 = jnp.concatenate([col, row, zi])
    w_j = [jnp.concatenate([wf[j], wr[j], zf]) for j in range(EFEAT)]

    ones_n = jnp.ones((NP, NFEAT), f32)
    spmv = _spmv_calls(NP, EP)

    degp = [spmv(ones_n, rows2, rows2, w_j[j]) for j in range(EFEAT)]

    BS = 256
    gn = NP // BS
    blk = lambda: pl.BlockSpec((BS, NFEAT), lambda i: (i, 0))
    blk_a = lambda: pl.BlockSpec((BS, NFEAT), lambda i: (i, 0))
    blk_b = lambda: pl.BlockSpec((BS, NFEAT), lambda i: (i + gn, 0))
    nshape = jax.ShapeDtypeStruct((NP, NFEAT), f32)

    dm0, dm1 = pl.pallas_call(
        _prep_body,
        grid=(gn,),
        in_specs=[blk_a(), blk_b(), blk_a(), blk_b()],
        out_specs=[blk(), blk()],
        out_shape=[nshape, nshape],
    )(degp[0], degp[0], degp[1], degp[1])

    h = jnp.zeros((NP, NFEAT), f32).at[:N].set(x[:, 4:4 + NFEAT])

    for l in range(NLAYERS):
        p = [spmv(h, rows2, cols2, w_j[j]) for j in range(EFEAT)]
        t0, t1 = pl.pallas_call(
            _stage_a_body,
            grid=(gn,),
            in_specs=[blk(), blk(), blk(), blk_a(), blk_b(), blk_a(), blk_b()],
            out_specs=[blk(), blk()],
            out_shape=[nshape, nshape],
        )(h, dm0, dm1, p[0], p[0], p[1], p[1])
        q = [spmv(t, rows2, cols2, w_j[j])
             for j, t in enumerate((t0, t1))]
        h = pl.pallas_call(
            functools.partial(_stage_b_body, relu=(l < NLAYERS - 1)),
            grid=(gn,),
            in_specs=[blk(), blk(), blk(), blk(), blk(),
                      blk_a(), blk_b(), blk_a(), blk_b(),
                      pl.BlockSpec((K, NFEAT, HIDDEN), lambda i: (0, 0, 0)),
                      pl.BlockSpec((K, NFEAT, HIDDEN), lambda i: (0, 0, 0)),
                      pl.BlockSpec((2, HIDDEN), lambda i: (0, 0))],
            out_specs=blk(),
            out_shape=nshape,
        )(h, t0, t1, dm0, dm1, q[0], q[0], q[1], q[1],
          cheb_W[l * EFEAT + 0], cheb_W[l * EFEAT + 1],
          cheb_b[l * EFEAT:l * EFEAT + 2])

    return h[:N]


# speedup vs baseline: 1.2848x; 1.2848x over previous
import functools
import jax
import jax.numpy as jnp
from jax import lax
from jax.experimental import pallas as pl
from jax.experimental.pallas import tpu as pltpu
from jax.experimental.pallas import tpu_sc as plsc

NFEAT = 128
HIDDEN = 128
NLAYERS = 3
K = 3
EFEAT = 2

_NW = 32
_C = 128


def _spmv_calls(NP, EP):
    per_tile = EP // _NW
    n_chunks = per_tile // _C
    rps = NP // 16
    mesh = plsc.VectorSubcoreMesh(core_axis_name="c", subcore_axis_name="s")

    @functools.partial(
        pl.kernel, mesh=mesh,
        out_type=jax.ShapeDtypeStruct((2 * NP, NFEAT), jnp.float32),
        scratch_types=[
            pltpu.VMEM((_C,), jnp.int32),
            pltpu.VMEM((_C,), jnp.int32),
            pltpu.VMEM((_C,), jnp.float32),
            pltpu.VMEM((_C, NFEAT), jnp.float32),
            pltpu.VMEM_SHARED((NP, NFEAT), jnp.float32),
            pltpu.SemaphoreType.DMA,
        ],
    )
    def spmv(y_h, g_h, s_h, w_h, out_h, gi_v, si_v, w_v, rows_v, acc, sem):
        cid = lax.axis_index("c")
        sid = lax.axis_index("s")
        wid = sid * 2 + cid

        def zrow(r, c):
            for f in range(NFEAT // 16):
                rows_v[r, pl.ds(f * 16, 16)] = jnp.zeros((16,), jnp.float32)
            return c

        lax.fori_loop(0, _C, zrow, 0)
        for kblk in range(rps // _C):
            pltpu.sync_copy(rows_v,
                            acc.at[pl.ds(sid * rps + kblk * _C, _C)])
        plsc.subcore_barrier()
        base0 = wid * per_tile

        def chunk(ci, carry):
            b = base0 + ci * _C
            pltpu.sync_copy(g_h.at[pl.ds(b, _C)], gi_v)
            pltpu.sync_copy(s_h.at[pl.ds(b, _C)], si_v)
            pltpu.sync_copy(w_h.at[pl.ds(b, _C)], w_v)
            pltpu.async_copy(y_h.at[gi_v], rows_v, sem).wait()

            def group(g, c2):
                wv = w_v[pl.ds(g * 16, 16)]
                for i in range(16):
                    ws = lax.gather(
                        wv, jnp.full((16, 1), i, jnp.int32),
                        lax.GatherDimensionNumbers(
                            offset_dims=(), collapsed_slice_dims=(0,),
                            start_index_map=(0,)),
                        slice_sizes=(1,),
                        mode=lax.GatherScatterMode.PROMISE_IN_BOUNDS)
                    e = g * 16 + i
                    for f in range(NFEAT // 16):
                        seg = rows_v[e, pl.ds(f * 16, 16)]
                        rows_v[e, pl.ds(f * 16, 16)] = seg * ws
                return c2

            lax.fori_loop(0, _C // 16, group, 0)
            pltpu.sync_copy(rows_v, acc.at[si_v], add=True)
            return carry

        lax.fori_loop(0, n_chunks, chunk, 0)
        plsc.subcore_barrier()
        pltpu.sync_copy(acc.at[pl.ds(sid * rps, rps)],
                        out_h.at[pl.ds(cid * NP + sid * rps, rps)])

    return spmv


def _flag_body(ei_ref, e0_ref, o_ref):
    i = pl.program_id(0)

    @pl.when(i == 0)
    def _():
        o_ref[...] = jnp.zeros_like(o_ref)

    r0 = e0_ref[0, 0]
    c0 = e0_ref[1, 0]
    hit = jnp.logical_and(ei_ref[0:1, :] == c0, ei_ref[1:2, :] == r0)
    hitf = jnp.max(hit.astype(jnp.float32), axis=1, keepdims=True)
    o_ref[...] = jnp.maximum(o_ref[...], hitf)


def _mlp_body(efT_ref, ei_ref, W1t_ref, b1_ref, W2t_ref, b2_ref, found_ref,
              wf_ref, wr_ref):
    ef = efT_ref[...]
    h = jnp.dot(W1t_ref[...], ef, preferred_element_type=jnp.float32)
    h = jnp.maximum(h + b1_ref[...], 0.0)
    t = jnp.dot(W2t_ref[...], h, preferred_element_type=jnp.float32)
    w = ef + t + b2_ref[...]
    loop = ei_ref[0:1, :] == ei_ref[1:2, :]
    w = jnp.where(loop, 0.0, w)
    wf_ref[...] = w
    wr_ref[...] = w * (1.0 - found_ref[0, 0])


def _prep_body(d0a, d0b, d1a, d1b, o0, o1):
    o0[...] = d0a[...] + d0b[...] - 1.0
    o1[...] = d1a[...] + d1b[...] - 1.0


def _stage_a_body(h, dm0, dm1, p0a, p0b, p1a, p1b, t0, t1):
    hv = h[...]
    t0[...] = dm0[...] * hv - p0a[...] - p0b[...]
    t1[...] = dm1[...] * hv - p1a[...] - p1b[...]


def _stage_b_body(h, t0, t1, dm0, dm1, q0a, q0b, q1a, q1b, W0, W1, bias, out,
                  *, relu):
    hv = h[...]
    t0v = t0[...]
    t1v = t1[...]
    x2_0 = 2.0 * (dm0[...] * t0v - q0a[...] - q0b[...]) - hv
    x2_1 = 2.0 * (dm1[...] * t1v - q1a[...] - q1b[...]) - hv
    dot = lambda a, b: jnp.dot(a, b, preferred_element_type=jnp.float32)
    acc = dot(hv, W0[0] + W1[0])
    acc = acc + dot(t0v, W0[1]) + dot(t1v, W1[1])
    acc = acc + dot(x2_0, W0[2]) + dot(x2_1, W1[2])
    acc = acc + bias[0:1, :] + bias[1:2, :]
    if relu:
        acc = jnp.maximum(acc, 0.0)
    out[...] = acc


def kernel(x, edge_attr, et_W1, et_b1, et_W2, et_b2, cheb_W, cheb_b, edge_index):
    N = x.shape[0]
    E = edge_index.shape[1]
    NP = ((N + 255) // 256) * 256
    EP = ((2 * E + _NW * _C - 1) // (_NW * _C)) * (_NW * _C)
    f32 = jnp.float32

    ei = edge_index.astype(jnp.int32)
    row, col = ei[0], ei[1]

    BE = 2560
    ge = E // BE
    e0 = ei[:, 0:1]
    found = pl.pallas_call(
        _flag_body,
        grid=(ge,),
        in_specs=[
            pl.BlockSpec((2, BE), lambda i: (0, i)),
            pl.BlockSpec(memory_space=pltpu.SMEM),
        ],
        out_specs=pl.BlockSpec((1, 1), lambda i: (0, 0)),
        out_shape=jax.ShapeDtypeStruct((1, 1), f32),
    )(ei, e0)

    efT = edge_attr[:, :EFEAT].T
    W1t = et_W1.T
    b1c = et_b1[:, None]
    W2t = et_W2.T
    b2c = et_b2[:, None]
    wf, wr = pl.pallas_call(
        _mlp_body,
        grid=(ge,),
        in_specs=[
            pl.BlockSpec((2, BE), lambda i: (0, i)),
            pl.BlockSpec((2, BE), lambda i: (0, i)),
            pl.BlockSpec((HIDDEN, 2), lambda i: (0, 0)),
            pl.BlockSpec((HIDDEN, 1), lambda i: (0, 0)),
            pl.BlockSpec((2, HIDDEN), lambda i: (0, 0)),
            pl.BlockSpec((2, 1), lambda i: (0, 0)),
            pl.BlockSpec(memory_space=pltpu.SMEM),
        ],
        out_specs=[
            pl.BlockSpec((2, BE), lambda i: (0, i)),
            pl.BlockSpec((2, BE), lambda i: (0, i)),
        ],
        out_shape=[
            jax.ShapeDtypeStruct((2, E), f32),
            jax.ShapeDtypeStruct((2, E), f32),
        ],
    )(efT, ei, W1t, b1c, W2t, b2c, found)

    pad = EP - 2 * E
    zi = jnp.zeros((pad,), jnp.int32)
    zf = jnp.zeros((pad,), f32)
    rows2 = jnp.concatenate([row, col, zi])
    cols2 = jnp.concatenate([col, row, zi])
    w_j = [jnp.concatenate([wf[j], wr[j], zf]) for j in range(EFEAT)]

    ones_n = jnp.ones((NP, NFEAT), f32)
    spmv = _spmv_calls(NP, EP)

    degp = [spmv(ones_n, rows2, rows2, w_j[j]) for j in range(EFEAT)]

    BS = 256
    gn = NP // BS
    blk = lambda: pl.BlockSpec((BS, NFEAT), lambda i: (i, 0))
    blk_a = lambda: pl.BlockSpec((BS, NFEAT), lambda i: (i, 0))
    blk_b = lambda: pl.BlockSpec((BS, NFEAT), lambda i: (i + gn, 0))
    nshape = jax.ShapeDtypeStruct((NP, NFEAT), f32)

    dm0, dm1 = pl.pallas_call(
        _prep_body,
        grid=(gn,),
        in_specs=[blk_a(), blk_b(), blk_a(), blk_b()],
        out_specs=[blk(), blk()],
        out_shape=[nshape, nshape],
    )(degp[0], degp[0], degp[1], degp[1])

    h = jnp.zeros((NP, NFEAT), f32).at[:N].set(x[:, 4:4 + NFEAT])

    for l in range(NLAYERS):
        p = [spmv(h, rows2, cols2, w_j[j]) for j in range(EFEAT)]
        t0, t1 = pl.pallas_call(
            _stage_a_body,
            grid=(gn,),
            in_specs=[blk(), blk(), blk(), blk_a(), blk_b(), blk_a(), blk_b()],
            out_specs=[blk(), blk()],
            out_shape=[nshape, nshape],
        )(h, dm0, dm1, p[0], p[0], p[1], p[1])
        q = [spmv(t, rows2, cols2, w_j[j])
             for j, t in enumerate((t0, t1))]
        h = pl.pallas_call(
            functools.partial(_stage_b_body, relu=(l < NLAYERS - 1)),
            grid=(gn,),
            in_specs=[blk(), blk(), blk(), blk(), blk(),
                      blk_a(), blk_b(), blk_a(), blk_b(),
                      pl.BlockSpec((K, NFEAT, HIDDEN), lambda i: (0, 0, 0)),
                      pl.BlockSpec((K, NFEAT, HIDDEN), lambda i: (0, 0, 0)),
                      pl.BlockSpec((2, HIDDEN), lambda i: (0, 0))],
            out_specs=blk(),
            out_shape=nshape,
        )(h, t0, t1, dm0, dm1, q[0], q[0], q[1], q[1],
          cheb_W[l * EFEAT + 0], cheb_W[l * EFEAT + 1],
          cheb_b[l * EFEAT:l * EFEAT + 2])

    return h[:N]
